# baseline (speedup 1.0000x reference)
import jax
import jax.numpy as jnp
from jax import lax
from jax.experimental import pallas as pl
from jax.experimental.pallas import tpu as pltpu
from jax.experimental.pallas import tpu_sc as plsc

N = 10000
E = 320000
D = 128

NC = 2
NS = 16
CH = 128
NCH = (E + NS * CH - 1) // (NS * CH)
EPT = NCH * CH
EPAD = NS * EPT - E
NP2 = 10240
HALF = 5120
NPH = 6144
RPT = NPH // NS
DRPT = NP2 // NS
EPS = 1e-5


def _degfilter_body(src_hbm, dst_hbm, deg_hbm, wsrc_hbm, wdst_hbm, cnt_hbm,
                    src_v, dst_v, wsrc_v, wdst_v, ones_v, zvec_v, cnt_v,
                    acc_sh):
    c = lax.axis_index("c")
    s = lax.axis_index("s")
    zero16 = jnp.zeros((16,), jnp.float32)
    one16 = jnp.ones((16,), jnp.float32)
    it = lax.iota(jnp.int32, 16)

    @pl.loop(0, DRPT // 16)
    def _zero(i):
        zvec_v[pl.ds(i * 16, 16)] = zero16

    for i in range(CH // 16):
        ones_v[pl.ds(i * 16, 16)] = one16

    pltpu.sync_copy(zvec_v, acc_sh.at[pl.ds(s * DRPT, DRPT)])
    pltpu.sync_copy(src_hbm.at[s], src_v)
    pltpu.sync_copy(dst_hbm.at[s], dst_v)

    svec = jnp.full((16,), s * 16, jnp.int32)
    trash_src = N + ((it + svec) & 127)
    trash_dst = HALF + ((it + svec) & 255)

    @pl.loop(0, EPT // 16)
    def _prefill(i):
        wsrc_v[pl.ds(i * 16, 16)] = trash_src
        wdst_v[pl.ds(i * 16, 16)] = trash_dst

    plsc.subcore_barrier()

    @pl.loop(0, NCH)
    def _scat(j):
        pltpu.sync_copy(ones_v, acc_sh.at[dst_v.at[j]], add=True)

    bvec = jnp.full((16,), c.astype(jnp.int32) * HALF, jnp.int32)

    @pl.loop(0, NCH, init_carry=jnp.int32(0))
    def _compact(j, off):
        for jj in range(CH // 16):
            sv = src_v[j, pl.ds(jj * 16, 16)]
            dv = dst_v[j, pl.ds(jj * 16, 16)]
            idx = dv - bvec
            valid = (idx >= 0) & (idx < HALF)
            vi = jnp.where(valid, jnp.full((16,), 1, jnp.int32),
                           jnp.full((16,), 0, jnp.int32))
            pos = plsc.cumsum(vi) + jnp.full((16,), off - 1, jnp.int32)
            plsc.store_scatter(wsrc_v, [pos], sv, mask=valid)
            plsc.store_scatter(wdst_v, [pos], idx, mask=valid)
            off = off + jnp.sum(vi)
        return off

    cnt = _compact
    cnt_v[pl.ds(0, 16)] = jnp.full((16,), cnt, jnp.int32)

    pltpu.sync_copy(wsrc_v, wsrc_hbm.at[c].at[s])
    pltpu.sync_copy(wdst_v, wdst_hbm.at[c].at[s])
    pltpu.sync_copy(cnt_v, cnt_hbm.at[c].at[s])

    plsc.subcore_barrier()
    pltpu.sync_copy(acc_sh.at[pl.ds(s * DRPT, DRPT)],
                    deg_hbm.at[c].at[pl.ds(s * DRPT, DRPT)])


def _conv_body(p_hbm, wsrc_hbm, wdst_hbm, cnt_hbm, out_hbm,
               wsrc_v, wdst_v, cnt_v, rows0, rows1, sem0, sem1,
               ssem0, ssem1, acc_sh):
    c = lax.axis_index("c")
    s = lax.axis_index("s")
    zero16 = jnp.zeros((16,), jnp.float32)

    @pl.loop(0, CH)
    def _zrow(i):
        for jj in range(D // 16):
            rows0[i, pl.ds(jj * 16, 16)] = zero16

    for k in range(RPT // CH):
        pltpu.sync_copy(rows0, acc_sh.at[pl.ds(s * RPT + k * CH, CH)])

    pltpu.sync_copy(wsrc_hbm.at[c].at[s], wsrc_v)
    pltpu.sync_copy(wdst_hbm.at[c].at[s], wdst_v)
    pltpu.sync_copy(cnt_hbm.at[c].at[s], cnt_v)

    cnt = cnt_v[pl.ds(0, 16)][0]
    ncd = jnp.maximum((cnt + CH - 1) // CH, 1)

    plsc.subcore_barrier()

    pltpu.async_copy(p_hbm.at[wsrc_v.at[0]], rows0, sem0)

    @pl.loop(0, ncd)
    def _pipe(j):
        even = lax.rem(j, 2) == 0

        @pl.when(even)
        def _do_even():
            pltpu.make_async_copy(p_hbm.at[wsrc_v.at[j]], rows0, sem0).wait()
            pltpu.async_copy(rows0, acc_sh.at[wdst_v.at[j]], ssem0, add=True)

            @pl.when(j + 1 < ncd)
            def _nxt():
                @pl.when(j > 0)
                def _drain():
                    pltpu.make_async_copy(
                        rows1, acc_sh.at[wdst_v.at[0]], ssem1).wait()

                pltpu.async_copy(p_hbm.at[wsrc_v.at[j + 1]], rows1, sem1)

        @pl.when(jnp.logical_not(even))
        def _do_odd():
            pltpu.make_async_copy(p_hbm.at[wsrc_v.at[j]], rows1, sem1).wait()
            pltpu.async_copy(rows1, acc_sh.at[wdst_v.at[j]], ssem1, add=True)

            @pl.when(j + 1 < ncd)
            def _nxt():
                pltpu.make_async_copy(
                    rows0, acc_sh.at[wdst_v.at[0]], ssem0).wait()
                pltpu.async_copy(p_hbm.at[wsrc_v.at[j + 1]], rows0, sem0)

    last_even = lax.rem(ncd - 1, 2) == 0

    @pl.when(jnp.logical_and(last_even, ncd > 1))
    def _dr1():
        pltpu.make_async_copy(rows1, acc_sh.at[wdst_v.at[0]], ssem1).wait()

    @pl.when(jnp.logical_and(jnp.logical_not(last_even), ncd > 1))
    def _dr2():
        pltpu.make_async_copy(rows0, acc_sh.at[wdst_v.at[0]], ssem0).wait()

    @pl.when(last_even)
    def _dr3():
        pltpu.make_async_copy(rows0, acc_sh.at[wdst_v.at[0]], ssem0).wait()

    @pl.when(jnp.logical_not(last_even))
    def _dr4():
        pltpu.make_async_copy(rows1, acc_sh.at[wdst_v.at[0]], ssem1).wait()

    plsc.subcore_barrier()
    pltpu.sync_copy(acc_sh.at[pl.ds(s * RPT, RPT)],
                    out_hbm.at[c].at[pl.ds(s * RPT, RPT)])


def _make_sc_calls():
    mesh = plsc.VectorSubcoreMesh(core_axis_name="c", subcore_axis_name="s")
    degfilter_call = pl.kernel(
        _degfilter_body,
        compiler_params=pltpu.CompilerParams(needs_layout_passes=False),
        out_type=(
            jax.ShapeDtypeStruct((NC, NP2), jnp.float32),
            jax.ShapeDtypeStruct((NC, NS, EPT), jnp.int32),
            jax.ShapeDtypeStruct((NC, NS, EPT), jnp.int32),
            jax.ShapeDtypeStruct((NC, NS, 16), jnp.int32),
        ),
        mesh=mesh,
        scratch_types=[
            pltpu.VMEM((NCH, CH), jnp.int32),
            pltpu.VMEM((NCH, CH), jnp.int32),
            pltpu.VMEM((EPT,), jnp.int32),
            pltpu.VMEM((EPT,), jnp.int32),
            pltpu.VMEM((CH,), jnp.float32),
            pltpu.VMEM((DRPT,), jnp.float32),
            pltpu.VMEM((16,), jnp.int32),
            pltpu.VMEM_SHARED((NP2,), jnp.float32),
        ],
    )
    conv_call = pl.kernel(
        _conv_body,
        out_type=jax.ShapeDtypeStruct((NC, NPH, D), jnp.float32),
        mesh=mesh,
        scratch_types=[
            pltpu.VMEM((NCH, CH), jnp.int32),
            pltpu.VMEM((NCH, CH), jnp.int32),
            pltpu.VMEM((16,), jnp.int32),
            pltpu.VMEM((CH, D), jnp.float32),
            pltpu.VMEM((CH, D), jnp.float32),
            pltpu.SemaphoreType.DMA,
            pltpu.SemaphoreType.DMA,
            pltpu.SemaphoreType.DMA,
            pltpu.SemaphoreType.DMA,
            pltpu.VMEM_SHARED((NPH, D), jnp.float32),
        ],
    )
    return degfilter_call, conv_call


def _dinv_from(degp_ref):
    deg = degp_ref[0, :N] + 1.0
    return lax.rsqrt(deg)[:, None]


def _stitch(s_ref):
    return jnp.concatenate([s_ref[0, :HALF, :], s_ref[1, :N - HALF, :]], axis=0)


def _tc1_body(x_ref, mask_ref, wenc_ref, w1_ref, dec_ref, degp_ref, p1_ref):
    dinv = _dinv_from(degp_ref)
    h = jnp.dot(x_ref[...], wenc_ref[...], preferred_element_type=jnp.float32)
    h = jnp.where(mask_ref[...] == 0, dec_ref[...], h)
    p1_ref[:N, :] = jnp.dot(h, w1_ref[...],
                            preferred_element_type=jnp.float32) * dinv
    p1_ref[N:, :] = jnp.zeros((NP2 - N, D), jnp.float32)


def _ln_prelu(s, g_ref, be_ref, alpha_ref):
    mu = jnp.mean(s, axis=-1, keepdims=True)
    xc = s - mu
    var = jnp.mean(xc * xc, axis=-1, keepdims=True)
    o = xc * lax.rsqrt(var + EPS) * g_ref[...] + be_ref[...]
    return jnp.where(o > 0, o, alpha_ref[0, 0] * o)


def _tc2_body(s_ref, p1_ref, degp_ref, b1_ref, g1_ref, be1_ref, alpha_ref,
              w2_ref, p2_ref):
    dinv = _dinv_from(degp_ref)
    t = (_stitch(s_ref) + p1_ref[:N, :]) * dinv + b1_ref[...]
    o = _ln_prelu(t, g1_ref, be1_ref, alpha_ref)
    p2_ref[:N, :] = jnp.dot(o, w2_ref[...],
                            preferred_element_type=jnp.float32) * dinv
    p2_ref[N:, :] = jnp.zeros((NP2 - N, D), jnp.float32)


def _tc3_body(s_ref, p2_ref, degp_ref, b2_ref, g2_ref, be2_ref, alpha_ref,
              wout_ref, bout_ref, out_ref):
    dinv = _dinv_from(degp_ref)
    t = (_stitch(s_ref) + p2_ref[:N, :]) * dinv + b2_ref[...]
    o = _ln_prelu(t, g2_ref, be2_ref, alpha_ref)
    out_ref[...] = jnp.dot(o, wout_ref[...],
                           preferred_element_type=jnp.float32) + bout_ref[...]


def kernel(x, edge_index, mask_vector, W_enc, dec_token, W1, b1, g1, beta1,
           alpha, W2, b2, g2, beta2, W_out, b_out):
    pad_idx = N + (jnp.arange(EPAD, dtype=jnp.int32) % (NP2 - N))
    src_p = jnp.concatenate([edge_index[0], pad_idx]).reshape(NS, NCH, CH)
    dst_p = jnp.concatenate([edge_index[1], pad_idx]).reshape(NS, NCH, CH)

    degfilter_call, conv_call = _make_sc_calls()

    degp, wsrc, wdst, cnt = degfilter_call(src_p, dst_p)
    wsrc = wsrc.reshape(NC, NS, NCH, CH)
    wdst = wdst.reshape(NC, NS, NCH, CH)

    p1 = pl.pallas_call(
        _tc1_body,
        out_shape=jax.ShapeDtypeStruct((NP2, D), jnp.float32),
    )(x, mask_vector, W_enc, W1, dec_token, degp)

    s1 = conv_call(p1, wsrc, wdst, cnt)

    b1r = b1.reshape(1, D)
    g1r = g1.reshape(1, D)
    be1r = beta1.reshape(1, D)
    alphar = alpha.reshape(1, 1)
    p2 = pl.pallas_call(
        _tc2_body,
        out_shape=jax.ShapeDtypeStruct((NP2, D), jnp.float32),
    )(s1, p1, degp, b1r, g1r, be1r, alphar, W2)

    s2 = conv_call(p2, wsrc, wdst, cnt)

    out = pl.pallas_call(
        _tc3_body,
        out_shape=jax.ShapeDtypeStruct((N, D), jnp.float32),
    )(s2, p2, degp, b2.reshape(1, D), g2.reshape(1, D), beta2.reshape(1, D),
      alphar, W_out, b_out.reshape(1, D))
    return out


# speedup vs baseline: 1.1450x; 1.1450x over previous
import jax
import jax.numpy as jnp
from jax import lax
from jax.experimental import pallas as pl
from jax.experimental.pallas import tpu as pltpu
from jax.experimental.pallas import tpu_sc as plsc

N = 10000
E = 320000
D = 128

NC = 2
NS = 16
CH = 128
NCH = (E + NS * CH - 1) // (NS * CH)
EPT = NCH * CH
EPAD = NS * EPT - E
NP2 = 10240
HALF = 5120
NPH = 6144
RPT = NPH // NS
DRPT = NP2 // NS
EPS = 1e-5


def _degfilter_body(src_hbm, dst_hbm, deg_hbm, wsrc_hbm, wdst_hbm, cnt_hbm,
                    src_v, dst_v, wsrc_v, wdst_v, ones_v, zvec_v, cnt_v,
                    acc_sh):
    c = lax.axis_index("c")
    s = lax.axis_index("s")
    zero16 = jnp.zeros((16,), jnp.float32)
    one16 = jnp.ones((16,), jnp.float32)
    it = lax.iota(jnp.int32, 16)

    @pl.loop(0, DRPT // 16)
    def _zero(i):
        zvec_v[pl.ds(i * 16, 16)] = zero16

    for i in range(CH // 16):
        ones_v[pl.ds(i * 16, 16)] = one16

    pltpu.sync_copy(zvec_v, acc_sh.at[pl.ds(s * DRPT, DRPT)])
    pltpu.sync_copy(src_hbm.at[s], src_v)
    pltpu.sync_copy(dst_hbm.at[s], dst_v)

    svec = jnp.full((16,), s * 16, jnp.int32)
    trash_src = N + ((it + svec) & 127)
    trash_dst = HALF + ((it + svec) & 255)

    @pl.loop(0, EPT // 16)
    def _prefill(i):
        wsrc_v[pl.ds(i * 16, 16)] = trash_src
        wdst_v[pl.ds(i * 16, 16)] = trash_dst

    plsc.subcore_barrier()

    @pl.loop(0, NCH)
    def _scat(j):
        pltpu.sync_copy(ones_v, acc_sh.at[dst_v.at[j]], add=True)

    bvec = jnp.full((16,), c.astype(jnp.int32) * HALF, jnp.int32)

    @pl.loop(0, NCH, init_carry=jnp.int32(0))
    def _compact(j, off):
        for jj in range(CH // 16):
            sv = src_v[j, pl.ds(jj * 16, 16)]
            dv = dst_v[j, pl.ds(jj * 16, 16)]
            idx = dv - bvec
            valid = (idx >= 0) & (idx < HALF)
            vi = jnp.where(valid, jnp.full((16,), 1, jnp.int32),
                           jnp.full((16,), 0, jnp.int32))
            pos = plsc.cumsum(vi) + jnp.full((16,), off - 1, jnp.int32)
            plsc.store_scatter(wsrc_v, [pos], sv, mask=valid)
            plsc.store_scatter(wdst_v, [pos], idx, mask=valid)
            off = off + jnp.sum(vi)
        return off

    cnt = _compact
    cnt_v[pl.ds(0, 16)] = jnp.full((16,), cnt, jnp.int32)

    pltpu.sync_copy(wsrc_v, wsrc_hbm.at[c].at[s])
    pltpu.sync_copy(wdst_v, wdst_hbm.at[c].at[s])
    pltpu.sync_copy(cnt_v, cnt_hbm.at[c].at[s])

    plsc.subcore_barrier()
    pltpu.sync_copy(acc_sh.at[pl.ds(s * DRPT, DRPT)],
                    deg_hbm.at[c].at[pl.ds(s * DRPT, DRPT)])


def _conv_body(p_hbm, wsrc_hbm, wdst_hbm, cnt_hbm, out_hbm,
               wsrc_v, wdst_v, cnt_v, rows0, rows1, sem0, sem1, acc_sh):
    c = lax.axis_index("c")
    s = lax.axis_index("s")
    zero16 = jnp.zeros((16,), jnp.float32)

    @pl.loop(0, CH)
    def _zrow(i):
        for jj in range(D // 16):
            rows0[i, pl.ds(jj * 16, 16)] = zero16

    for k in range(RPT // CH):
        pltpu.sync_copy(rows0, acc_sh.at[pl.ds(s * RPT + k * CH, CH)])

    pltpu.sync_copy(wsrc_hbm.at[c].at[s], wsrc_v)
    pltpu.sync_copy(wdst_hbm.at[c].at[s], wdst_v)
    pltpu.sync_copy(cnt_hbm.at[c].at[s], cnt_v)

    cnt = cnt_v[pl.ds(0, 16)][0]
    ncd = jnp.maximum((cnt + CH - 1) // CH, 1)

    plsc.subcore_barrier()

    pltpu.async_copy(p_hbm.at[wsrc_v.at[0]], rows0, sem0)

    @pl.loop(0, ncd)
    def _pipe(j):
        even = lax.rem(j, 2) == 0

        @pl.when(j + 1 < ncd)
        def _nxt():
            @pl.when(even)
            def _():
                pltpu.async_copy(p_hbm.at[wsrc_v.at[j + 1]], rows1, sem1)

            @pl.when(jnp.logical_not(even))
            def _():
                pltpu.async_copy(p_hbm.at[wsrc_v.at[j + 1]], rows0, sem0)

        @pl.when(even)
        def _do_even():
            pltpu.make_async_copy(p_hbm.at[wsrc_v.at[j]], rows0, sem0).wait()
            pltpu.sync_copy(rows0, acc_sh.at[wdst_v.at[j]], add=True)

        @pl.when(jnp.logical_not(even))
        def _do_odd():
            pltpu.make_async_copy(p_hbm.at[wsrc_v.at[j]], rows1, sem1).wait()
            pltpu.sync_copy(rows1, acc_sh.at[wdst_v.at[j]], add=True)

    plsc.subcore_barrier()
    pltpu.sync_copy(acc_sh.at[pl.ds(s * RPT, RPT)],
                    out_hbm.at[c].at[pl.ds(s * RPT, RPT)])


def _make_sc_calls():
    mesh = plsc.VectorSubcoreMesh(core_axis_name="c", subcore_axis_name="s")
    degfilter_call = pl.kernel(
        _degfilter_body,
        compiler_params=pltpu.CompilerParams(needs_layout_passes=False),
        out_type=(
            jax.ShapeDtypeStruct((NC, NP2), jnp.float32),
            jax.ShapeDtypeStruct((NC, NS, EPT), jnp.int32),
            jax.ShapeDtypeStruct((NC, NS, EPT), jnp.int32),
            jax.ShapeDtypeStruct((NC, NS, 16), jnp.int32),
        ),
        mesh=mesh,
        scratch_types=[
            pltpu.VMEM((NCH, CH), jnp.int32),
            pltpu.VMEM((NCH, CH), jnp.int32),
            pltpu.VMEM((EPT,), jnp.int32),
            pltpu.VMEM((EPT,), jnp.int32),
            pltpu.VMEM((CH,), jnp.float32),
            pltpu.VMEM((DRPT,), jnp.float32),
            pltpu.VMEM((16,), jnp.int32),
            pltpu.VMEM_SHARED((NP2,), jnp.float32),
        ],
    )
    conv_call = pl.kernel(
        _conv_body,
        out_type=jax.ShapeDtypeStruct((NC, NPH, D), jnp.float32),
        mesh=mesh,
        scratch_types=[
            pltpu.VMEM((NCH, CH), jnp.int32),
            pltpu.VMEM((NCH, CH), jnp.int32),
            pltpu.VMEM((16,), jnp.int32),
            pltpu.VMEM((CH, D), jnp.float32),
            pltpu.VMEM((CH, D), jnp.float32),
            pltpu.SemaphoreType.DMA,
            pltpu.SemaphoreType.DMA,
            pltpu.VMEM_SHARED((NPH, D), jnp.float32),
        ],
    )
    return degfilter_call, conv_call


def _dinv_from(degp_ref):
    deg = degp_ref[0, :N] + 1.0
    return lax.rsqrt(deg)[:, None]


def _stitch(s_ref):
    return jnp.concatenate([s_ref[0, :HALF, :], s_ref[1, :N - HALF, :]], axis=0)


def _tc1_body(x_ref, mask_ref, wenc_ref, w1_ref, dec_ref, degp_ref, p1_ref):
    dinv = _dinv_from(degp_ref)
    h = jnp.dot(x_ref[...], wenc_ref[...], preferred_element_type=jnp.float32)
    h = jnp.where(mask_ref[...] == 0, dec_ref[...], h)
    p1_ref[:N, :] = jnp.dot(h, w1_ref[...],
                            preferred_element_type=jnp.float32) * dinv
    p1_ref[N:, :] = jnp.zeros((NP2 - N, D), jnp.float32)


def _ln_prelu(s, g_ref, be_ref, alpha_ref):
    mu = jnp.mean(s, axis=-1, keepdims=True)
    xc = s - mu
    var = jnp.mean(xc * xc, axis=-1, keepdims=True)
    o = xc * lax.rsqrt(var + EPS) * g_ref[...] + be_ref[...]
    return jnp.where(o > 0, o, alpha_ref[0, 0] * o)


def _tc2_body(s_ref, p1_ref, degp_ref, b1_ref, g1_ref, be1_ref, alpha_ref,
              w2_ref, p2_ref):
    dinv = _dinv_from(degp_ref)
    t = (_stitch(s_ref) + p1_ref[:N, :]) * dinv + b1_ref[...]
    o = _ln_prelu(t, g1_ref, be1_ref, alpha_ref)
    p2_ref[:N, :] = jnp.dot(o, w2_ref[...],
                            preferred_element_type=jnp.float32) * dinv
    p2_ref[N:, :] = jnp.zeros((NP2 - N, D), jnp.float32)


def _tc3_body(s_ref, p2_ref, degp_ref, b2_ref, g2_ref, be2_ref, alpha_ref,
              wout_ref, bout_ref, out_ref):
    dinv = _dinv_from(degp_ref)
    t = (_stitch(s_ref) + p2_ref[:N, :]) * dinv + b2_ref[...]
    o = _ln_prelu(t, g2_ref, be2_ref, alpha_ref)
    out_ref[...] = jnp.dot(o, wout_ref[...],
                           preferred_element_type=jnp.float32) + bout_ref[...]


def kernel(x, edge_index, mask_vector, W_enc, dec_token, W1, b1, g1, beta1,
           alpha, W2, b2, g2, beta2, W_out, b_out):
    pad_idx = N + (jnp.arange(EPAD, dtype=jnp.int32) % (NP2 - N))
    src_p = jnp.concatenate([edge_index[0], pad_idx]).reshape(NS, NCH, CH)
    dst_p = jnp.concatenate([edge_index[1], pad_idx]).reshape(NS, NCH, CH)

    degfilter_call, conv_call = _make_sc_calls()

    degp, wsrc, wdst, cnt = degfilter_call(src_p, dst_p)
    wsrc = wsrc.reshape(NC, NS, NCH, CH)
    wdst = wdst.reshape(NC, NS, NCH, CH)

    p1 = pl.pallas_call(
        _tc1_body,
        out_shape=jax.ShapeDtypeStruct((NP2, D), jnp.float32),
    )(x, mask_vector, W_enc, W1, dec_token, degp)

    s1 = conv_call(p1, wsrc, wdst, cnt)

    b1r = b1.reshape(1, D)
    g1r = g1.reshape(1, D)
    be1r = beta1.reshape(1, D)
    alphar = alpha.reshape(1, 1)
    p2 = pl.pallas_call(
        _tc2_body,
        out_shape=jax.ShapeDtypeStruct((NP2, D), jnp.float32),
    )(s1, p1, degp, b1r, g1r, be1r, alphar, W2)

    s2 = conv_call(p2, wsrc, wdst, cnt)

    out = pl.pallas_call(
        _tc3_body,
        out_shape=jax.ShapeDtypeStruct((N, D), jnp.float32),
    )(s2, p2, degp, b2.reshape(1, D), g2.reshape(1, D), beta2.reshape(1, D),
      alphar, W_out, b_out.reshape(1, D))
    return out


# speedup vs baseline: 1.2204x; 1.0659x over previous
import jax
import jax.numpy as jnp
from jax import lax
from jax.experimental import pallas as pl
from jax.experimental.pallas import tpu as pltpu
from jax.experimental.pallas import tpu_sc as plsc

N = 10000
E = 320000
D = 128

NC = 2
NS = 16
CH = 128
NCH = (E + NS * CH - 1) // (NS * CH)
EPT = NCH * CH
EPAD = NS * EPT - E
NP2 = 10240
HALF = 5120
NPH = 6144
RPT = NPH // NS
DRPT = NP2 // NS
EPS = 1e-5


def _degfilter_body(src_hbm, dst_hbm, deg_hbm, wsrc_hbm, wdst_hbm, cnt_hbm,
                    src_v, dst_v, wsrc_v, wdst_v, ones_v, zvec_v, cnt_v,
                    dsem, acc_sh):
    c = lax.axis_index("c")
    s = lax.axis_index("s")
    zero16 = jnp.zeros((16,), jnp.float32)
    one16 = jnp.ones((16,), jnp.float32)
    it = lax.iota(jnp.int32, 16)

    @pl.loop(0, DRPT // 16)
    def _zero(i):
        zvec_v[pl.ds(i * 16, 16)] = zero16

    for i in range(CH // 16):
        ones_v[pl.ds(i * 16, 16)] = one16

    pltpu.sync_copy(zvec_v, acc_sh.at[pl.ds(s * DRPT, DRPT)])
    pltpu.sync_copy(src_hbm.at[s], src_v)
    pltpu.sync_copy(dst_hbm.at[s], dst_v)

    svec = jnp.full((16,), s * 16, jnp.int32)
    trash_src = N + ((it + svec) & 127)
    trash_dst = HALF + ((it + svec) & 255)

    plsc.subcore_barrier()

    bvec = jnp.full((16,), c.astype(jnp.int32) * HALF, jnp.int32)

    @pl.loop(0, NCH, init_carry=jnp.int32(0))
    def _compact(j, off):
        pltpu.async_copy(ones_v, acc_sh.at[dst_v.at[j]], dsem, add=True)
        for jj in range(CH // 16):
            sv = src_v[j, pl.ds(jj * 16, 16)]
            dv = dst_v[j, pl.ds(jj * 16, 16)]
            idx = dv - bvec
            valid = (idx >= 0) & (idx < HALF)
            vi = jnp.where(valid, jnp.full((16,), 1, jnp.int32),
                           jnp.full((16,), 0, jnp.int32))
            pos = plsc.cumsum(vi) + jnp.full((16,), off - 1, jnp.int32)
            plsc.store_scatter(wsrc_v, [pos], sv, mask=valid)
            plsc.store_scatter(wdst_v, [pos], idx, mask=valid)
            off = off + jnp.sum(vi)

        @pl.when(j >= 8)
        def _drain_one():
            pltpu.make_async_copy(ones_v, acc_sh.at[dst_v.at[0]], dsem).wait()

        return off

    cnt = _compact
    for _ in range(8):
        pltpu.make_async_copy(ones_v, acc_sh.at[dst_v.at[0]], dsem).wait()

    limit = jnp.maximum(((cnt + CH - 1) // CH) * CH, CH)
    limitv = jnp.full((16,), limit, jnp.int32)
    for k in range(CH // 16):
        pos = it + jnp.full((16,), cnt + 16 * k, jnp.int32)
        m = pos < limitv
        plsc.store_scatter(wsrc_v, [pos], trash_src, mask=m)
        plsc.store_scatter(wdst_v, [pos], trash_dst, mask=m)

    cnt_v[pl.ds(0, 16)] = jnp.full((16,), cnt, jnp.int32)

    pltpu.sync_copy(wsrc_v, wsrc_hbm.at[c].at[s])
    pltpu.sync_copy(wdst_v, wdst_hbm.at[c].at[s])
    pltpu.sync_copy(cnt_v, cnt_hbm.at[c].at[s])

    plsc.subcore_barrier()
    pltpu.sync_copy(acc_sh.at[pl.ds(s * DRPT, DRPT)],
                    deg_hbm.at[c].at[pl.ds(s * DRPT, DRPT)])


def _conv_body(p_hbm, wsrc_hbm, wdst_hbm, cnt_hbm, out_hbm,
               wsrc_v, wdst_v, cnt_v, rows0, rows1, sem0, sem1, acc_sh):
    c = lax.axis_index("c")
    s = lax.axis_index("s")
    zero16 = jnp.zeros((16,), jnp.float32)

    @pl.loop(0, CH)
    def _zrow(i):
        for jj in range(D // 16):
            rows0[i, pl.ds(jj * 16, 16)] = zero16

    for k in range(RPT // CH):
        pltpu.sync_copy(rows0, acc_sh.at[pl.ds(s * RPT + k * CH, CH)])

    pltpu.sync_copy(wsrc_hbm.at[c].at[s], wsrc_v)
    pltpu.sync_copy(wdst_hbm.at[c].at[s], wdst_v)
    pltpu.sync_copy(cnt_hbm.at[c].at[s], cnt_v)

    cnt = cnt_v[pl.ds(0, 16)][0]
    ncd = jnp.maximum((cnt + CH - 1) // CH, 1)

    plsc.subcore_barrier()

    pltpu.async_copy(p_hbm.at[wsrc_v.at[0]], rows0, sem0)

    @pl.loop(0, ncd)
    def _pipe(j):
        even = lax.rem(j, 2) == 0

        @pl.when(j + 1 < ncd)
        def _nxt():
            @pl.when(even)
            def _():
                pltpu.async_copy(p_hbm.at[wsrc_v.at[j + 1]], rows1, sem1)

            @pl.when(jnp.logical_not(even))
            def _():
                pltpu.async_copy(p_hbm.at[wsrc_v.at[j + 1]], rows0, sem0)

        @pl.when(even)
        def _do_even():
            pltpu.make_async_copy(p_hbm.at[wsrc_v.at[j]], rows0, sem0).wait()
            pltpu.sync_copy(rows0, acc_sh.at[wdst_v.at[j]], add=True)

        @pl.when(jnp.logical_not(even))
        def _do_odd():
            pltpu.make_async_copy(p_hbm.at[wsrc_v.at[j]], rows1, sem1).wait()
            pltpu.sync_copy(rows1, acc_sh.at[wdst_v.at[j]], add=True)

    plsc.subcore_barrier()
    pltpu.sync_copy(acc_sh.at[pl.ds(s * RPT, RPT)],
                    out_hbm.at[c].at[pl.ds(s * RPT, RPT)])


def _make_sc_calls():
    mesh = plsc.VectorSubcoreMesh(core_axis_name="c", subcore_axis_name="s")
    degfilter_call = pl.kernel(
        _degfilter_body,
        compiler_params=pltpu.CompilerParams(needs_layout_passes=False),
        out_type=(
            jax.ShapeDtypeStruct((NC, NP2), jnp.float32),
            jax.ShapeDtypeStruct((NC, NS, EPT), jnp.int32),
            jax.ShapeDtypeStruct((NC, NS, EPT), jnp.int32),
            jax.ShapeDtypeStruct((NC, NS, 16), jnp.int32),
        ),
        mesh=mesh,
        scratch_types=[
            pltpu.VMEM((NCH, CH), jnp.int32),
            pltpu.VMEM((NCH, CH), jnp.int32),
            pltpu.VMEM((EPT,), jnp.int32),
            pltpu.VMEM((EPT,), jnp.int32),
            pltpu.VMEM((CH,), jnp.float32),
            pltpu.VMEM((DRPT,), jnp.float32),
            pltpu.VMEM((16,), jnp.int32),
            pltpu.SemaphoreType.DMA,
            pltpu.VMEM_SHARED((NP2,), jnp.float32),
        ],
    )
    conv_call = pl.kernel(
        _conv_body,
        out_type=jax.ShapeDtypeStruct((NC, NPH, D), jnp.float32),
        mesh=mesh,
        scratch_types=[
            pltpu.VMEM((NCH, CH), jnp.int32),
            pltpu.VMEM((NCH, CH), jnp.int32),
            pltpu.VMEM((16,), jnp.int32),
            pltpu.VMEM((CH, D), jnp.float32),
            pltpu.VMEM((CH, D), jnp.float32),
            pltpu.SemaphoreType.DMA,
            pltpu.SemaphoreType.DMA,
            pltpu.VMEM_SHARED((NPH, D), jnp.float32),
        ],
    )
    return degfilter_call, conv_call


def _dinv_from(degp_ref):
    deg = degp_ref[0, :N] + 1.0
    return lax.rsqrt(deg)[:, None]


def _stitch(s_ref):
    return jnp.concatenate([s_ref[0, :HALF, :], s_ref[1, :N - HALF, :]], axis=0)


def _tc1_body(x_ref, mask_ref, wenc_ref, w1_ref, dec_ref, degp_ref, p1_ref):
    dinv = _dinv_from(degp_ref)
    h = jnp.dot(x_ref[...], wenc_ref[...], preferred_element_type=jnp.float32)
    h = jnp.where(mask_ref[...] == 0, dec_ref[...], h)
    p1_ref[:N, :] = jnp.dot(h, w1_ref[...],
                            preferred_element_type=jnp.float32) * dinv
    p1_ref[N:, :] = jnp.zeros((NP2 - N, D), jnp.float32)


def _ln_prelu(s, g_ref, be_ref, alpha_ref):
    mu = jnp.mean(s, axis=-1, keepdims=True)
    xc = s - mu
    var = jnp.mean(xc * xc, axis=-1, keepdims=True)
    o = xc * lax.rsqrt(var + EPS) * g_ref[...] + be_ref[...]
    return jnp.where(o > 0, o, alpha_ref[0, 0] * o)


def _tc2_body(s_ref, p1_ref, degp_ref, b1_ref, g1_ref, be1_ref, alpha_ref,
              w2_ref, p2_ref):
    dinv = _dinv_from(degp_ref)
    t = (_stitch(s_ref) + p1_ref[:N, :]) * dinv + b1_ref[...]
    o = _ln_prelu(t, g1_ref, be1_ref, alpha_ref)
    p2_ref[:N, :] = jnp.dot(o, w2_ref[...],
                            preferred_element_type=jnp.float32) * dinv
    p2_ref[N:, :] = jnp.zeros((NP2 - N, D), jnp.float32)


def _tc3_body(s_ref, p2_ref, degp_ref, b2_ref, g2_ref, be2_ref, alpha_ref,
              wout_ref, bout_ref, out_ref):
    dinv = _dinv_from(degp_ref)
    t = (_stitch(s_ref) + p2_ref[:N, :]) * dinv + b2_ref[...]
    o = _ln_prelu(t, g2_ref, be2_ref, alpha_ref)
    out_ref[...] = jnp.dot(o, wout_ref[...],
                           preferred_element_type=jnp.float32) + bout_ref[...]


def kernel(x, edge_index, mask_vector, W_enc, dec_token, W1, b1, g1, beta1,
           alpha, W2, b2, g2, beta2, W_out, b_out):
    pad_idx = N + (jnp.arange(EPAD, dtype=jnp.int32) % (NP2 - N))
    src_p = jnp.concatenate([edge_index[0], pad_idx]).reshape(NS, NCH, CH)
    dst_p = jnp.concatenate([edge_index[1], pad_idx]).reshape(NS, NCH, CH)

    degfilter_call, conv_call = _make_sc_calls()

    degp, wsrc, wdst, cnt = degfilter_call(src_p, dst_p)
    wsrc = wsrc.reshape(NC, NS, NCH, CH)
    wdst = wdst.reshape(NC, NS, NCH, CH)

    p1 = pl.pallas_call(
        _tc1_body,
        out_shape=jax.ShapeDtypeStruct((NP2, D), jnp.float32),
    )(x, mask_vector, W_enc, W1, dec_token, degp)

    s1 = conv_call(p1, wsrc, wdst, cnt)

    b1r = b1.reshape(1, D)
    g1r = g1.reshape(1, D)
    be1r = beta1.reshape(1, D)
    alphar = alpha.reshape(1, 1)
    p2 = pl.pallas_call(
        _tc2_body,
        out_shape=jax.ShapeDtypeStruct((NP2, D), jnp.float32),
    )(s1, p1, degp, b1r, g1r, be1r, alphar, W2)

    s2 = conv_call(p2, wsrc, wdst, cnt)

    out = pl.pallas_call(
        _tc3_body,
        out_shape=jax.ShapeDtypeStruct((N, D), jnp.float32),
    )(s2, p2, degp, b2.reshape(1, D), g2.reshape(1, D), beta2.reshape(1, D),
      alphar, W_out, b_out.reshape(1, D))
    return out
